# baseline (speedup 1.0000x reference)
import functools

import jax
import jax.numpy as jnp
from jax import lax
from jax.experimental import pallas as pl
from jax.experimental.pallas import tpu as pltpu
from jax.experimental.pallas import tpu_sc as plsc

_INFO = plsc.get_sparse_core_info()
_NC = _INFO.num_cores
_NS = _INFO.num_subcores
_L = _INFO.num_lanes
_NW = _NC * _NS

_B = 16384
_D = 64
_W = 128
_BPW = _B // _NW
_CHUNK = 256
_NCHUNK = _BPW // _CHUNK


def _neg_sqrt(x):
    y = (x + jnp.float32(16.0)) * jnp.float32(0.125)
    for _ in range(6):
        y = jnp.float32(0.5) * (y + x / y)
    return -y


def _score_body(head_hbm, rel_hbm, tail_hbm, ent_hbm, relt_hbm, out_hbm,
                hidx, ridx, tidx, hbuf, rbuf, tbuf, rot, asm, outv, sem):
    wid = lax.axis_index("s") * _NC + lax.axis_index("c")
    base = wid * _BPW

    pltpu.sync_copy(head_hbm.at[pl.ds(base, _BPW)], hidx)
    pltpu.sync_copy(rel_hbm.at[pl.ds(base, _BPW)], ridx)
    pltpu.sync_copy(tail_hbm.at[pl.ds(base, _BPW)], tidx)

    for c in range(_NCHUNK):
        isl = pl.ds(c * _CHUNK, _CHUNK)
        ch = pltpu.async_copy(ent_hbm.at[hidx.at[isl]], hbuf, sem)
        cr = pltpu.async_copy(relt_hbm.at[ridx.at[isl]], rbuf, sem)
        ct = pltpu.async_copy(ent_hbm.at[tidx.at[isl]], tbuf, sem)
        ch.wait()
        cr.wait()
        ct.wait()

        def group(g, _):
            r0 = g * _L
            for j in range(_L):
                r = r0 + j
                s = jnp.zeros((_L,), jnp.float32)
                for k in range(_D // _L):
                    sl = pl.ds(k * _L, _L)
                    diff = hbuf[r, sl] + rbuf[r, sl] - tbuf[r, sl]
                    s = s + diff * diff
                for shift in (8, 4, 2, 1):
                    rot[j, pl.ds(0, _L)] = s
                    s = s + rot[j, pl.ds(shift, _L)]
                asm[pl.ds(j, _L)] = s
            sums = asm[pl.ds(0, _L)]
            outv[pl.ds(c * _CHUNK + r0, _L)] = _neg_sqrt(sums)
            return 0

        lax.fori_loop(0, _CHUNK // _L, group, 0)

    pltpu.sync_copy(outv, out_hbm.at[pl.ds(base, _BPW)])


@functools.partial(
    pl.kernel,
    mesh=plsc.VectorSubcoreMesh(core_axis_name="c", subcore_axis_name="s"),
    out_type=jax.ShapeDtypeStruct((_B,), jnp.float32),
    scratch_types=[
        pltpu.VMEM((_BPW,), jnp.int32),
        pltpu.VMEM((_BPW,), jnp.int32),
        pltpu.VMEM((_BPW,), jnp.int32),
        pltpu.VMEM((_CHUNK, _W), jnp.float32),
        pltpu.VMEM((_CHUNK, _W), jnp.float32),
        pltpu.VMEM((_CHUNK, _W), jnp.float32),
        pltpu.VMEM((_L, _L + 8), jnp.float32),
        pltpu.VMEM((2 * _L,), jnp.float32),
        pltpu.VMEM((_BPW,), jnp.float32),
        pltpu.SemaphoreType.DMA,
    ],
    compiler_params=pltpu.CompilerParams(use_tc_tiling_on_sc=True),
)
def _transe_score(*refs):
    _score_body(*refs)


def kernel(head, relation, tail, entity_table, relation_table):
    head = head.astype(jnp.int32)
    relation = relation.astype(jnp.int32)
    tail = tail.astype(jnp.int32)
    ent2 = jnp.pad(entity_table, ((0, 0), (0, _W - _D)))
    rel2 = jnp.pad(relation_table, ((0, 0), (0, _W - _D)))
    return _transe_score(head, relation, tail, ent2, rel2)


# speedup vs baseline: 2.4991x; 2.4991x over previous
import functools

import jax
import jax.numpy as jnp
from jax import lax
from jax.experimental import pallas as pl
from jax.experimental.pallas import tpu as pltpu
from jax.experimental.pallas import tpu_sc as plsc

_INFO = plsc.get_sparse_core_info()
_NC = _INFO.num_cores
_NS = _INFO.num_subcores
_L = _INFO.num_lanes
_NW = _NC * _NS

_B = 16384
_D = 64
_W = 128
_BPW = _B // _NW
_CHUNK = 256
_NCHUNK = _BPW // _CHUNK

_NE = 1000000
_BKE = 8192
_NBLK = (_NE + 2 * _BKE - 1) // (2 * _BKE)
_ROWS = _NBLK * _BKE
_NLANEBLK = (_NE + _BKE - 1) // _BKE - 1

_NR = 1000
_RH = _NR // 2


def _relayout_body(a_ref, b_ref, o_ref):
    x = jnp.concatenate([a_ref[...], b_ref[...]], axis=0)
    o_ref[...] = jnp.transpose(x)


_relayout = pl.pallas_call(
    _relayout_body,
    grid=(_NBLK,),
    in_specs=[
        pl.BlockSpec((_D, _BKE), lambda j: (0, 2 * j)),
        pl.BlockSpec((_D, _BKE), lambda j: (0, jnp.minimum(2 * j + 1, _NLANEBLK))),
    ],
    out_specs=pl.BlockSpec((_BKE, _W), lambda j: (j, 0)),
    out_shape=jax.ShapeDtypeStruct((_ROWS, _W), jnp.float32),
)


def _neg_sqrt(x):
    y = (x + jnp.float32(16.0)) * jnp.float32(0.125)
    for _ in range(6):
        y = jnp.float32(0.5) * (y + x / y)
    return -y


def _score_body(head_hbm, hsel_hbm, rel_hbm, rsel_hbm, tail_hbm, tsel_hbm,
                ent_hbm, relt_hbm, out_hbm,
                hidx, hsel, ridx, rsel, tidx, tsel,
                hbuf, rbuf, tbuf, rot, asm, outv, sem):
    wid = lax.axis_index("s") * _NC + lax.axis_index("c")
    base = wid * _BPW

    pltpu.sync_copy(head_hbm.at[pl.ds(base, _BPW)], hidx)
    pltpu.sync_copy(rel_hbm.at[pl.ds(base, _BPW)], ridx)
    pltpu.sync_copy(tail_hbm.at[pl.ds(base, _BPW)], tidx)
    pltpu.sync_copy(hsel_hbm.at[pl.ds(base, _BPW)], hsel.at[pl.ds(0, _BPW)])
    pltpu.sync_copy(rsel_hbm.at[pl.ds(base, _BPW)], rsel.at[pl.ds(0, _BPW)])
    pltpu.sync_copy(tsel_hbm.at[pl.ds(base, _BPW)], tsel.at[pl.ds(0, _BPW)])

    for c in range(_NCHUNK):
        isl = pl.ds(c * _CHUNK, _CHUNK)
        ch = pltpu.async_copy(ent_hbm.at[hidx.at[isl]], hbuf, sem)
        cr = pltpu.async_copy(relt_hbm.at[ridx.at[isl]], rbuf, sem)
        ct = pltpu.async_copy(ent_hbm.at[tidx.at[isl]], tbuf, sem)
        ch.wait()
        cr.wait()
        ct.wait()

        def group(g, _):
            r0 = g * _L
            for j in range(_L):
                r = r0 + j
                ri = c * _CHUNK + r
                bh = hsel[pl.ds(ri, _L)][0]
                br = rsel[pl.ds(ri, _L)][0]
                bt = tsel[pl.ds(ri, _L)][0]
                s = jnp.zeros((_L,), jnp.float32)
                for k in range(_D // _L):
                    o = k * _L
                    diff = (hbuf[r, pl.ds(bh + o, _L)]
                            + rbuf[r, pl.ds(br + o, _L)]
                            - tbuf[r, pl.ds(bt + o, _L)])
                    s = s + diff * diff
                for shift in (8, 4, 2, 1):
                    rot[j, pl.ds(0, _L)] = s
                    s = s + rot[j, pl.ds(shift, _L)]
                asm[pl.ds(j, _L)] = s
            sums = asm[pl.ds(0, _L)]
            outv[pl.ds(c * _CHUNK + r0, _L)] = _neg_sqrt(sums)
            return 0

        lax.fori_loop(0, _CHUNK // _L, group, 0)

    pltpu.sync_copy(outv, out_hbm.at[pl.ds(base, _BPW)])


@functools.partial(
    pl.kernel,
    mesh=plsc.VectorSubcoreMesh(core_axis_name="c", subcore_axis_name="s"),
    out_type=jax.ShapeDtypeStruct((_B,), jnp.float32),
    scratch_types=[
        pltpu.VMEM((_BPW,), jnp.int32),
        pltpu.VMEM((_BPW + _L,), jnp.int32),
        pltpu.VMEM((_BPW,), jnp.int32),
        pltpu.VMEM((_BPW + _L,), jnp.int32),
        pltpu.VMEM((_BPW,), jnp.int32),
        pltpu.VMEM((_BPW + _L,), jnp.int32),
        pltpu.VMEM((_CHUNK, _W), jnp.float32),
        pltpu.VMEM((_CHUNK, _W), jnp.float32),
        pltpu.VMEM((_CHUNK, _W), jnp.float32),
        pltpu.VMEM((_L, _L + 8), jnp.float32),
        pltpu.VMEM((2 * _L,), jnp.float32),
        pltpu.VMEM((_BPW,), jnp.float32),
        pltpu.SemaphoreType.DMA,
    ],
    compiler_params=pltpu.CompilerParams(use_tc_tiling_on_sc=True),
)
def _transe_score(*refs):
    _score_body(*refs)


def _entity_coords(i):
    row = (i >> 14) * _BKE + (i & (_BKE - 1))
    sel = ((i >> 13) & 1) * _D
    return row, sel


def kernel(head, relation, tail, entity_table, relation_table):
    head = head.astype(jnp.int32)
    relation = relation.astype(jnp.int32)
    tail = tail.astype(jnp.int32)

    ent_t = entity_table.T
    ent2 = _relayout(ent_t, ent_t)
    rel2 = jnp.concatenate(
        [relation_table[:_RH], relation_table[_RH:]], axis=1)

    hrow, hsel = _entity_coords(head)
    trow, tsel = _entity_coords(tail)
    rrow = jnp.where(relation >= _RH, relation - _RH, relation)
    rsel = jnp.where(relation >= _RH, _D, 0).astype(jnp.int32)

    return _transe_score(hrow, hsel, rrow, rsel, trow, tsel, ent2, rel2)


# speedup vs baseline: 2.6748x; 1.0703x over previous
import functools

import jax
import jax.numpy as jnp
from jax import lax
from jax.experimental import pallas as pl
from jax.experimental.pallas import tpu as pltpu
from jax.experimental.pallas import tpu_sc as plsc

_INFO = plsc.get_sparse_core_info()
_NC = _INFO.num_cores
_NS = _INFO.num_subcores
_L = _INFO.num_lanes
_NW = _NC * _NS

_B = 16384
_D = 64
_W = 128
_BPW = _B // _NW
_CHUNK = 128
_NCHUNK = _BPW // _CHUNK

_NE = 1000000
_BKE = 16384
_NBLK = (_NE + 2 * _BKE - 1) // (2 * _BKE)
_ROWS = _NBLK * _BKE
_NLANEBLK = (_NE + _BKE - 1) // _BKE - 1
_SH = _BKE.bit_length() - 1

_NR = 1000
_RH = _NR // 2


def _relayout_body(a_ref, b_ref, o_ref):
    x = jnp.concatenate([a_ref[...], b_ref[...]], axis=0)
    o_ref[...] = jnp.transpose(x)


_relayout = pl.pallas_call(
    _relayout_body,
    grid=(_NBLK,),
    in_specs=[
        pl.BlockSpec((_D, _BKE), lambda j: (0, 2 * j)),
        pl.BlockSpec((_D, _BKE), lambda j: (0, jnp.minimum(2 * j + 1, _NLANEBLK))),
    ],
    out_specs=pl.BlockSpec((_BKE, _W), lambda j: (j, 0)),
    out_shape=jax.ShapeDtypeStruct((_ROWS, _W), jnp.float32),
    compiler_params=pltpu.CompilerParams(
        dimension_semantics=("arbitrary",)),
)


def _neg_sqrt(x):
    y = (x + jnp.float32(16.0)) * jnp.float32(0.125)
    for _ in range(6):
        y = jnp.float32(0.5) * (y + x / y)
    return -y


def _score_body(head_hbm, rel_hbm, tail_hbm, ent_hbm, relt_hbm, out_hbm,
                hidx, hsel, ridx, rsel, tidx, tsel,
                bufs0, bufs1, rot, asm, outv, sem0, sem1):
    wid = lax.axis_index("s") * _NC + lax.axis_index("c")
    base = wid * _BPW

    pltpu.sync_copy(head_hbm.at[pl.ds(base, _BPW)], hidx)
    pltpu.sync_copy(rel_hbm.at[pl.ds(base, _BPW)], ridx)
    pltpu.sync_copy(tail_hbm.at[pl.ds(base, _BPW)], tidx)

    def conv(i, _):
        sl = pl.ds(i * _L, _L)
        for raw, sel in ((hidx, hsel), (tidx, tsel)):
            v = raw[sl]
            sel[sl] = lax.shift_right_logical(v, _SH) & 1
            raw[sl] = lax.shift_right_logical(v, _SH + 1) * _BKE \
                + (v & (_BKE - 1))
        rv = ridx[sl]
        big = rv >= _RH
        rsel[sl] = jnp.where(big, 1, 0)
        ridx[sl] = jnp.where(big, rv - _RH, rv)
        return 0

    lax.fori_loop(0, _BPW // _L, conv, 0)

    bufsets = (bufs0, bufs1)
    sems = (sem0, sem1)

    def start(c):
        s = bufsets[c % 2]
        pltpu.async_copy(ent_hbm.at[hidx.at[pl.ds(c * _CHUNK, _CHUNK)]],
                         s.at[0], sems[c % 2])
        pltpu.async_copy(relt_hbm.at[ridx.at[pl.ds(c * _CHUNK, _CHUNK)]],
                         s.at[1], sems[c % 2])
        pltpu.async_copy(ent_hbm.at[tidx.at[pl.ds(c * _CHUNK, _CHUNK)]],
                         s.at[2], sems[c % 2])

    def drain(c):
        s = bufsets[c % 2]
        pltpu.make_async_copy(ent_hbm.at[hidx.at[pl.ds(c * _CHUNK, _CHUNK)]],
                              s.at[0], sems[c % 2]).wait()
        pltpu.make_async_copy(relt_hbm.at[ridx.at[pl.ds(c * _CHUNK, _CHUNK)]],
                              s.at[1], sems[c % 2]).wait()
        pltpu.make_async_copy(ent_hbm.at[tidx.at[pl.ds(c * _CHUNK, _CHUNK)]],
                              s.at[2], sems[c % 2]).wait()

    start(0)
    for c in range(_NCHUNK):
        if c + 1 < _NCHUNK:
            start(c + 1)
        drain(c)
        bufs = bufsets[c % 2]

        def group(g, _):
            r0 = g * _L
            for j in range(_L):
                r = r0 + j
                ri = c * _CHUNK + r
                bh = hsel[pl.ds(ri, _L)][0] * _D
                br = rsel[pl.ds(ri, _L)][0] * _D
                bt = tsel[pl.ds(ri, _L)][0] * _D
                s = jnp.zeros((_L,), jnp.float32)
                for k in range(_D // _L):
                    o = k * _L
                    diff = (bufs[0, r, pl.ds(bh + o, _L)]
                            + bufs[1, r, pl.ds(br + o, _L)]
                            - bufs[2, r, pl.ds(bt + o, _L)])
                    s = s + diff * diff
                for shift in (8, 4, 2, 1):
                    rot[j, pl.ds(0, _L)] = s
                    s = s + rot[j, pl.ds(shift, _L)]
                asm[pl.ds(j, _L)] = s
            sums = asm[pl.ds(0, _L)]
            outv[pl.ds(c * _CHUNK + r0, _L)] = _neg_sqrt(sums)
            return 0

        lax.fori_loop(0, _CHUNK // _L, group, 0)

    pltpu.sync_copy(outv, out_hbm.at[pl.ds(base, _BPW)])


@functools.partial(
    pl.kernel,
    mesh=plsc.VectorSubcoreMesh(core_axis_name="c", subcore_axis_name="s"),
    out_type=jax.ShapeDtypeStruct((_B,), jnp.float32),
    scratch_types=[
        pltpu.VMEM((_BPW,), jnp.int32),
        pltpu.VMEM((_BPW + _L,), jnp.int32),
        pltpu.VMEM((_BPW,), jnp.int32),
        pltpu.VMEM((_BPW + _L,), jnp.int32),
        pltpu.VMEM((_BPW,), jnp.int32),
        pltpu.VMEM((_BPW + _L,), jnp.int32),
        pltpu.VMEM((3, _CHUNK, _W), jnp.float32),
        pltpu.VMEM((3, _CHUNK, _W), jnp.float32),
        pltpu.VMEM((_L, _L + 8), jnp.float32),
        pltpu.VMEM((2 * _L,), jnp.float32),
        pltpu.VMEM((_BPW,), jnp.float32),
        pltpu.SemaphoreType.DMA,
        pltpu.SemaphoreType.DMA,
    ],
    compiler_params=pltpu.CompilerParams(use_tc_tiling_on_sc=True),
)
def _transe_score(*refs):
    _score_body(*refs)


def kernel(head, relation, tail, entity_table, relation_table):
    head = head.astype(jnp.int32)
    relation = relation.astype(jnp.int32)
    tail = tail.astype(jnp.int32)
    ent_t = entity_table.T
    ent2 = _relayout(ent_t, ent_t)
    rel2 = jnp.concatenate(
        [relation_table[:_RH], relation_table[_RH:]], axis=1)
    return _transe_score(head, relation, tail, ent2, rel2)
